# baseline (speedup 1.0000x reference)
import functools
import jax
import jax.numpy as jnp
from jax import lax
from jax.experimental import pallas as pl
from jax.experimental.pallas import tpu as pltpu
from jax.experimental.pallas import tpu_sc as plsc

_R = 16


def _sc_body(x_hbm, pos_hbm, out_hbm, xv, pv, nc, nw, s_len):
    wid = lax.axis_index("s") * nc + lax.axis_index("c")
    total_rows = x_hbm.shape[0]
    rows_per_w = total_rows // nw
    base = wid * rows_per_w
    pos_base = lax.rem(base, s_len)
    n_chunks = rows_per_w // _R

    def chunk_body(c, _):
        row0 = base + c * _R
        prow0 = pos_base + c * _R
        pltpu.sync_copy(x_hbm.at[pl.ds(row0, _R)], xv)
        pltpu.sync_copy(pos_hbm.at[pl.ds(prow0, _R)], pv)

        def row_body(r, _):
            def vec_body(j, _):
                sl = pl.ds(j * 16, 16)
                xv[r, sl] = xv[r, sl] + pv[r, sl]
                return 0

            return lax.fori_loop(0, xv.shape[1] // 16, vec_body, 0)

        lax.fori_loop(0, _R, row_body, 0)
        pltpu.sync_copy(xv, out_hbm.at[pl.ds(row0, _R)])
        return 0

    lax.fori_loop(0, n_chunks, chunk_body, 0)


def kernel(x, pos_table):
    B, S, D = x.shape
    pos = pos_table[:S]
    x2 = x.reshape(B * S, D)
    info = plsc.get_sparse_core_info()
    nc, ns = info.num_cores, info.num_subcores
    nw = nc * ns
    mesh = plsc.VectorSubcoreMesh(core_axis_name="c", subcore_axis_name="s")
    body = functools.partial(_sc_body, nc=nc, nw=nw, s_len=S)
    run = pl.kernel(
        body,
        out_type=jax.ShapeDtypeStruct((B * S, D), x.dtype),
        mesh=mesh,
        scratch_types=[
            pltpu.VMEM((_R, D), jnp.float32),
            pltpu.VMEM((_R, D), jnp.float32),
        ],
    )
    return run(x2, pos).reshape(B, S, D)


# speedup vs baseline: 2.5228x; 2.5228x over previous
import functools
import jax
import jax.numpy as jnp
from jax import lax
from jax.experimental import pallas as pl
from jax.experimental.pallas import tpu as pltpu
from jax.experimental.pallas import tpu_sc as plsc

_R = 16


def _sc_body(x_hbm, pos_hbm, out_hbm, xv0, xv1, pv0, pv1,
             sx0, sx1, sp0, sp1, so0, so1, nc, nw, s_len):
    wid = lax.axis_index("s") * nc + lax.axis_index("c")
    total_rows = x_hbm.shape[0]
    rows_per_w = total_rows // nw
    base = wid * rows_per_w
    pos_base = lax.rem(base, s_len)
    n_chunks = rows_per_w // _R
    d = x_hbm.shape[1]
    vecs = _R * d // 16

    xv = (xv0, xv1)
    pv = (pv0, pv1)
    sx = (sx0, sx1)
    sp = (sp0, sp1)
    so = (so0, so1)

    def start_in(c):
        b = c % 2
        dx = pltpu.async_copy(x_hbm.at[pl.ds(base + c * _R, _R)], xv[b], sx[b])
        dp = pltpu.async_copy(pos_hbm.at[pl.ds(pos_base + c * _R, _R)], pv[b], sp[b])
        return dx, dp

    descs_in = [None] * n_chunks
    descs_out = [None] * n_chunks
    descs_in[0] = start_in(0)

    for c in range(n_chunks):
        b = c % 2
        dx, dp = descs_in[c]
        dx.wait()
        dp.wait()
        if c >= 1:
            descs_out[c - 1].wait()
        if c + 1 < n_chunks:
            descs_in[c + 1] = start_in(c + 1)

        xb, pb = xv[b], pv[b]

        @plsc.parallel_loop(0, vecs, 1, unroll=8)
        def _vec(k):
            r = lax.shift_right_logical(k, 6)
            col = pl.multiple_of(lax.shift_left(lax.bitwise_and(k, 63), 4), 16)
            plsc.addupdate(xb.at[r, pl.ds(col, 16)], pb[r, pl.ds(col, 16)])

        descs_out[c] = pltpu.async_copy(
            xb, out_hbm.at[pl.ds(base + c * _R, _R)], so[b])

    descs_out[n_chunks - 1].wait()


def kernel(x, pos_table):
    B, S, D = x.shape
    pos = pos_table[:S]
    x2 = x.reshape(B * S, D)
    info = plsc.get_sparse_core_info()
    nc, ns = info.num_cores, info.num_subcores
    nw = nc * ns
    mesh = plsc.VectorSubcoreMesh(core_axis_name="c", subcore_axis_name="s")
    body = functools.partial(_sc_body, nc=nc, nw=nw, s_len=S)
    run = pl.kernel(
        body,
        out_type=jax.ShapeDtypeStruct((B * S, D), x.dtype),
        mesh=mesh,
        scratch_types=[
            pltpu.VMEM((_R, D), jnp.float32),
            pltpu.VMEM((_R, D), jnp.float32),
            pltpu.VMEM((_R, D), jnp.float32),
            pltpu.VMEM((_R, D), jnp.float32),
            pltpu.SemaphoreType.DMA,
            pltpu.SemaphoreType.DMA,
            pltpu.SemaphoreType.DMA,
            pltpu.SemaphoreType.DMA,
            pltpu.SemaphoreType.DMA,
            pltpu.SemaphoreType.DMA,
        ],
    )
    return run(x2, pos).reshape(B, S, D)
